# Initial kernel scaffold; baseline (speedup 1.0000x reference)
import jax
import jax.numpy as jnp
from jax.experimental import pallas as pl


def kernel(x, t, W, b, time_embed):
    raise NotImplementedError("write your pallas kernel here")


# speedup vs baseline: 2.0318x; 2.0318x over previous
import jax
import jax.numpy as jnp
from jax.experimental import pallas as pl
from jax.experimental.pallas import tpu as pltpu

N = 50000
D_IN = 256
D_OUT = 256
N_STEPS = 1000
ROWS = 1000


def _fused_kernel(x_ref, t_ref, wt_ref, b_ref, emb_ref, o_ref):
    acc = jnp.dot(x_ref[...], wt_ref[...], preferred_element_type=jnp.float32)
    acc = acc + b_ref[...]
    idx = t_ref[0, 0, :]
    steps = jax.lax.broadcasted_iota(jnp.int32, (ROWS, N_STEPS), 1)
    onehot = (steps == idx[:, None]).astype(jnp.bfloat16)
    gamma = jnp.dot(onehot, emb_ref[...], preferred_element_type=jnp.float32)
    o_ref[...] = jax.nn.softplus(gamma * acc)


def kernel(x, t, W, b, time_embed):
    t3 = t.astype(jnp.int32).reshape(N // ROWS, 1, ROWS)
    wt = W.T
    b2 = b.reshape(1, D_OUT)
    emb16 = time_embed.astype(jnp.bfloat16)
    grid = (N // ROWS,)
    return pl.pallas_call(
        _fused_kernel,
        grid=grid,
        in_specs=[
            pl.BlockSpec((ROWS, D_IN), lambda i: (i, 0)),
            pl.BlockSpec((1, 1, ROWS), lambda i: (i, 0, 0)),
            pl.BlockSpec((D_IN, D_OUT), lambda i: (0, 0)),
            pl.BlockSpec((1, D_OUT), lambda i: (0, 0)),
            pl.BlockSpec((N_STEPS, D_OUT), lambda i: (0, 0)),
        ],
        out_specs=pl.BlockSpec((ROWS, D_OUT), lambda i: (i, 0)),
        out_shape=jax.ShapeDtypeStruct((N, D_OUT), jnp.float32),
        compiler_params=pltpu.CompilerParams(
            dimension_semantics=("arbitrary",),
        ),
    )(x, t3, wt, b2, emb16)


# speedup vs baseline: 2.0406x; 1.0043x over previous
import jax
import jax.numpy as jnp
from jax.experimental import pallas as pl
from jax.experimental.pallas import tpu as pltpu

N = 50000
D_IN = 256
D_OUT = 256
N_STEPS = 1000
ROWS = 1000


def _fused_kernel(x_ref, t_ref, wt_ref, b_ref, emb_ref, o_ref):
    acc = jnp.dot(x_ref[...].astype(jnp.bfloat16), wt_ref[...],
                  preferred_element_type=jnp.float32)
    acc = acc + b_ref[...]
    idx = t_ref[0, 0, :].astype(jnp.int16)
    steps = jax.lax.broadcasted_iota(jnp.int16, (ROWS, N_STEPS), 1)
    onehot = jnp.where(steps == idx[:, None],
                       jnp.bfloat16(1.0), jnp.bfloat16(0.0))
    gamma = jnp.dot(onehot, emb_ref[...], preferred_element_type=jnp.float32)
    o_ref[...] = jax.nn.softplus(gamma * acc)


def kernel(x, t, W, b, time_embed):
    t3 = t.astype(jnp.int32).reshape(N // ROWS, 1, ROWS)
    wt = W.T.astype(jnp.bfloat16)
    b2 = b.reshape(1, D_OUT)
    emb16 = time_embed.astype(jnp.bfloat16)
    grid = (N // ROWS,)
    return pl.pallas_call(
        _fused_kernel,
        grid=grid,
        in_specs=[
            pl.BlockSpec((ROWS, D_IN), lambda i: (i, 0)),
            pl.BlockSpec((1, 1, ROWS), lambda i: (i, 0, 0)),
            pl.BlockSpec((D_IN, D_OUT), lambda i: (0, 0)),
            pl.BlockSpec((1, D_OUT), lambda i: (0, 0)),
            pl.BlockSpec((N_STEPS, D_OUT), lambda i: (0, 0)),
        ],
        out_specs=pl.BlockSpec((ROWS, D_OUT), lambda i: (i, 0)),
        out_shape=jax.ShapeDtypeStruct((N, D_OUT), jnp.float32),
        compiler_params=pltpu.CompilerParams(
            dimension_semantics=("arbitrary",),
        ),
    )(x, t3, wt, b2, emb16)


# speedup vs baseline: 2.3148x; 1.1344x over previous
import jax
import jax.numpy as jnp
from jax.experimental import pallas as pl
from jax.experimental.pallas import tpu as pltpu

N = 50000
D_IN = 256
D_OUT = 256
N_STEPS = 1000
ROWS = 2000


def _fused_kernel(x_ref, t_ref, wt_ref, b_ref, emb_ref, o_ref):
    acc = jnp.dot(x_ref[...].astype(jnp.bfloat16), wt_ref[...],
                  preferred_element_type=jnp.float32)
    acc = acc + b_ref[...]
    idx = t_ref[0, 0, :].astype(jnp.int16)
    steps = jax.lax.broadcasted_iota(jnp.int16, (ROWS, N_STEPS), 1)
    onehot = jnp.where(steps == idx[:, None],
                       jnp.bfloat16(1.0), jnp.bfloat16(0.0))
    gamma = jnp.dot(onehot, emb_ref[...], preferred_element_type=jnp.float32)
    o_ref[...] = jax.nn.softplus(gamma * acc)


def kernel(x, t, W, b, time_embed):
    t3 = t.astype(jnp.int32).reshape(N // ROWS, 1, ROWS)
    wt = W.T.astype(jnp.bfloat16)
    b2 = b.reshape(1, D_OUT)
    emb16 = time_embed.astype(jnp.bfloat16)
    grid = (N // ROWS,)
    return pl.pallas_call(
        _fused_kernel,
        grid=grid,
        in_specs=[
            pl.BlockSpec((ROWS, D_IN), lambda i: (i, 0)),
            pl.BlockSpec((1, 1, ROWS), lambda i: (i, 0, 0)),
            pl.BlockSpec((D_IN, D_OUT), lambda i: (0, 0)),
            pl.BlockSpec((1, D_OUT), lambda i: (0, 0)),
            pl.BlockSpec((N_STEPS, D_OUT), lambda i: (0, 0)),
        ],
        out_specs=pl.BlockSpec((ROWS, D_OUT), lambda i: (i, 0)),
        out_shape=jax.ShapeDtypeStruct((N, D_OUT), jnp.float32),
        compiler_params=pltpu.CompilerParams(
            dimension_semantics=("arbitrary",),
        ),
    )(x, t3, wt, b2, emb16)


# speedup vs baseline: 2.3396x; 1.0107x over previous
import jax
import jax.numpy as jnp
from jax.experimental import pallas as pl
from jax.experimental.pallas import tpu as pltpu

N = 50000
D_IN = 256
D_OUT = 256
N_STEPS = 1000
ROWS = 5000


def _fused_kernel(x_ref, t_ref, wt_ref, b_ref, emb_ref, o_ref):
    acc = jnp.dot(x_ref[...].astype(jnp.bfloat16), wt_ref[...],
                  preferred_element_type=jnp.float32)
    acc = acc + b_ref[...]
    idx = t_ref[0, 0, :].astype(jnp.int16)
    steps = jax.lax.broadcasted_iota(jnp.int16, (ROWS, N_STEPS), 1)
    onehot = jnp.where(steps == idx[:, None],
                       jnp.bfloat16(1.0), jnp.bfloat16(0.0))
    gamma = jnp.dot(onehot, emb_ref[...], preferred_element_type=jnp.float32)
    o_ref[...] = jax.nn.softplus(gamma * acc)


def kernel(x, t, W, b, time_embed):
    t3 = t.astype(jnp.int32).reshape(N // ROWS, 1, ROWS)
    wt = W.T.astype(jnp.bfloat16)
    b2 = b.reshape(1, D_OUT)
    emb16 = time_embed.astype(jnp.bfloat16)
    grid = (N // ROWS,)
    return pl.pallas_call(
        _fused_kernel,
        grid=grid,
        in_specs=[
            pl.BlockSpec((ROWS, D_IN), lambda i: (i, 0)),
            pl.BlockSpec((1, 1, ROWS), lambda i: (i, 0, 0)),
            pl.BlockSpec((D_IN, D_OUT), lambda i: (0, 0)),
            pl.BlockSpec((1, D_OUT), lambda i: (0, 0)),
            pl.BlockSpec((N_STEPS, D_OUT), lambda i: (0, 0)),
        ],
        out_specs=pl.BlockSpec((ROWS, D_OUT), lambda i: (i, 0)),
        out_shape=jax.ShapeDtypeStruct((N, D_OUT), jnp.float32),
        compiler_params=pltpu.CompilerParams(
            dimension_semantics=("arbitrary",),
        ),
    )(x, t3, wt, b2, emb16)


# speedup vs baseline: 2.6713x; 1.1418x over previous
import jax
import jax.numpy as jnp
from jax.experimental import pallas as pl
from jax.experimental.pallas import tpu as pltpu

N = 50000
D_IN = 256
D_OUT = 256
N_STEPS = 1000
ROWS = 5000


def _fused_kernel(x_ref, t_ref, wt_ref, b_ref, emb_ref, o_ref):
    acc = jnp.dot(x_ref[...].astype(jnp.bfloat16), wt_ref[...],
                  preferred_element_type=jnp.float32)
    acc = acc + b_ref[...]
    idx = t_ref[0, 0, :].astype(jnp.int16)
    steps = jax.lax.broadcasted_iota(jnp.int16, (ROWS, N_STEPS), 1)
    onehot = jnp.where(steps == idx[:, None],
                       jnp.bfloat16(1.0), jnp.bfloat16(0.0))
    gamma = jnp.dot(onehot, emb_ref[...], preferred_element_type=jnp.float32)
    u = (gamma * acc) * jnp.float32(1.4426950408889634)
    m = jnp.maximum(u, 0.0)
    e = jnp.exp2(u - (m + m))
    o_ref[...] = (m + jnp.log2(1.0 + e)) * jnp.float32(0.6931471805599453)


def kernel(x, t, W, b, time_embed):
    t3 = t.astype(jnp.int32).reshape(N // ROWS, 1, ROWS)
    wt = W.T.astype(jnp.bfloat16)
    b2 = b.reshape(1, D_OUT)
    emb16 = time_embed.astype(jnp.bfloat16)
    grid = (N // ROWS,)
    return pl.pallas_call(
        _fused_kernel,
        grid=grid,
        in_specs=[
            pl.BlockSpec((ROWS, D_IN), lambda i: (i, 0)),
            pl.BlockSpec((1, 1, ROWS), lambda i: (i, 0, 0)),
            pl.BlockSpec((D_IN, D_OUT), lambda i: (0, 0)),
            pl.BlockSpec((1, D_OUT), lambda i: (0, 0)),
            pl.BlockSpec((N_STEPS, D_OUT), lambda i: (0, 0)),
        ],
        out_specs=pl.BlockSpec((ROWS, D_OUT), lambda i: (i, 0)),
        out_shape=jax.ShapeDtypeStruct((N, D_OUT), jnp.float32),
        compiler_params=pltpu.CompilerParams(
            dimension_semantics=("arbitrary",),
        ),
    )(x, t3, wt, b2, emb16)
